# baseline (speedup 1.0000x reference)
import functools

import jax
import jax.numpy as jnp
from jax import lax
from jax.experimental import pallas as pl
from jax.experimental.pallas import tpu as pltpu
from jax.experimental.pallas import tpu_sc as plsc

NUM_FIELDS = 26
VOCAB = 100000
HIDDEN = 64
BATCH = 16384

NC, NS = 2, 16
NW = NC * NS
Q = 32
TOTAL_Q = BATCH * NUM_FIELDS * (HIDDEN // Q)
ROWS_PER_W = TOTAL_Q // NW
CHUNK = 128
NCHUNK = ROWS_PER_W // CHUNK
NBUF = 8


@functools.partial(
    pl.kernel,
    out_type=jax.ShapeDtypeStruct((TOTAL_Q, Q), jnp.float32),
    mesh=plsc.VectorSubcoreMesh(core_axis_name="c", subcore_axis_name="s"),
    scratch_types=[
        pltpu.VMEM((NCHUNK, CHUNK), jnp.int32),
        pltpu.VMEM((NBUF, CHUNK, Q), jnp.float32),
        pltpu.SemaphoreType.DMA((NBUF,)),
        pltpu.SemaphoreType.DMA((NBUF,)),
    ],
    compiler_params=pltpu.CompilerParams(use_tc_tiling_on_sc=False),
)
def _gather_kernel(idx_hbm, table_hbm, out_hbm, idx_v, bufs, gsem, wsem):
    wid = lax.axis_index("s") * NC + lax.axis_index("c")
    base = wid * ROWS_PER_W
    pltpu.sync_copy(idx_hbm.at[wid], idx_v)

    def gather(j, b):
        return pltpu.make_async_copy(
            table_hbm.at[idx_v.at[j]], bufs.at[b], gsem.at[b])

    def write(j, b):
        return pltpu.make_async_copy(
            bufs.at[b], out_hbm.at[pl.ds(base + j * CHUNK, CHUNK)], wsem.at[b])

    for b in range(NBUF):
        gather(b, b).start()

    @pl.loop(0, NCHUNK - NBUF, step=NBUF)
    def _group(j0):
        for b in range(NBUF):
            gather(j0 + b, b).wait()
            write(j0 + b, b).start()
        for b in range(NBUF):
            write(j0 + b, b).wait()
            gather(j0 + b + NBUF, b).start()

    j0 = NCHUNK - NBUF
    for b in range(NBUF):
        gather(j0 + b, b).wait()
        write(j0 + b, b).start()
    for b in range(NBUF):
        write(j0 + b, b).wait()


def kernel(x_n_cat, tables):
    offsets = (jnp.arange(NUM_FIELDS, dtype=jnp.int32) * VOCAB)[None, :]
    flat = x_n_cat + offsets
    idx2 = 2 * flat[:, :, None] + jnp.arange(2, dtype=jnp.int32)
    idx = idx2.reshape(NW, NCHUNK, CHUNK)
    tp = jnp.concatenate([tables[:, 0::2, :], tables[:, 1::2, :]], axis=2)
    t32 = tp.reshape(NUM_FIELDS * VOCAB * 2, Q)
    out = _gather_kernel(idx, t32)
    return out.reshape(BATCH, NUM_FIELDS * HIDDEN)


# speedup vs baseline: 27.3324x; 27.3324x over previous
import functools

import jax
import jax.numpy as jnp
from jax import lax
from jax.experimental import pallas as pl
from jax.experimental.pallas import tpu as pltpu
from jax.experimental.pallas import tpu_sc as plsc

NUM_FIELDS = 26
VOCAB = 100000
HIDDEN = 64
BATCH = 16384

NC, NS = 2, 16
NW = NC * NS
Q = 32
TOTAL_Q = BATCH * NUM_FIELDS * (HIDDEN // Q)
ROWS_PER_W = TOTAL_Q // NW
CHUNK = 128
NCHUNK = ROWS_PER_W // CHUNK
NBUF = 8


@functools.partial(
    pl.kernel,
    out_type=jax.ShapeDtypeStruct((TOTAL_Q, Q), jnp.float32),
    mesh=plsc.VectorSubcoreMesh(core_axis_name="c", subcore_axis_name="s"),
    scratch_types=[
        pltpu.VMEM((NCHUNK, CHUNK), jnp.int32),
        pltpu.VMEM((NBUF, CHUNK, Q), jnp.float32),
        pltpu.SemaphoreType.DMA((NBUF,)),
        pltpu.SemaphoreType.DMA((NBUF,)),
    ],
    compiler_params=pltpu.CompilerParams(use_tc_tiling_on_sc=False),
)
def _gather_kernel(idx_hbm, table_hbm, out_hbm, idx_v, bufs, gsem, wsem):
    wid = lax.axis_index("s") * NC + lax.axis_index("c")
    base = wid * ROWS_PER_W
    pltpu.sync_copy(idx_hbm.at[wid], idx_v)

    def gather(j, b):
        return pltpu.make_async_copy(
            table_hbm.at[idx_v.at[j]], bufs.at[b], gsem.at[b])

    def write(j, b):
        return pltpu.make_async_copy(
            bufs.at[b], out_hbm.at[pl.ds(base + j * CHUNK, CHUNK)], wsem.at[b])

    for b in range(NBUF):
        gather(b, b).start()

    @pl.loop(0, NCHUNK - NBUF, step=NBUF)
    def _group(j0):
        for b in range(NBUF):
            gather(j0 + b, b).wait()
            write(j0 + b, b).start()
        for b in range(NBUF):
            write(j0 + b, b).wait()
            gather(j0 + b + NBUF, b).start()

    j0 = NCHUNK - NBUF
    for b in range(NBUF):
        gather(j0 + b, b).wait()
        write(j0 + b, b).start()
    for b in range(NBUF):
        write(j0 + b, b).wait()


def kernel(x_n_cat, tables):
    t13 = jnp.swapaxes(tables, 1, 2).reshape(NUM_FIELDS // 2, 2 * HIDDEN, VOCAB)
    tpk = jnp.swapaxes(t13, 1, 2)
    t32 = tpk.reshape(VOCAB * NUM_FIELDS * 2, Q)
    c = jnp.arange(2 * NUM_FIELDS, dtype=jnp.int32)
    cbase = (4 * VOCAB * (c // 4) + 2 * ((c // 2) % 2) + c % 2)[None, :]
    idx52 = 4 * jnp.repeat(x_n_cat, 2, axis=1) + cbase
    idx = idx52.reshape(NW, NCHUNK, CHUNK)
    out = _gather_kernel(idx, t32)
    return out.reshape(BATCH, NUM_FIELDS * HIDDEN)


# speedup vs baseline: 41.3307x; 1.5121x over previous
import functools

import jax
import jax.numpy as jnp
from jax import lax
from jax.experimental import pallas as pl
from jax.experimental.pallas import tpu as pltpu
from jax.experimental.pallas import tpu_sc as plsc

NUM_FIELDS = 26
VOCAB = 100000
HIDDEN = 64
BATCH = 16384

NC, NS = 2, 16
NW = NC * NS
Q = 32
NPAIR = NUM_FIELDS // 2
TOTAL_Q = BATCH * NUM_FIELDS * (HIDDEN // Q)
ROWS_PER_W = TOTAL_Q // NW
CHUNK = 128
NCHUNK = ROWS_PER_W // CHUNK
NBUF = 8
B_PER_W = BATCH // NW
QP_PER_B = 4 * NPAIR
QP_PER_I = 8 * NPAIR * 4
DIV_M = (1 << 24) // QP_PER_I + 1


@functools.partial(
    pl.kernel,
    out_type=jax.ShapeDtypeStruct((TOTAL_Q, Q), jnp.float32),
    mesh=plsc.VectorSubcoreMesh(core_axis_name="c", subcore_axis_name="s"),
    scratch_types=[
        pltpu.VMEM((B_PER_W, NUM_FIELDS), jnp.int32),
        pltpu.VMEM((NBUF, CHUNK), jnp.int32),
        pltpu.VMEM((NBUF, CHUNK, Q), jnp.float32),
        pltpu.SemaphoreType.DMA((NBUF,)),
        pltpu.SemaphoreType.DMA((NBUF,)),
    ],
    compiler_params=pltpu.CompilerParams(
        use_tc_tiling_on_sc=False, needs_layout_passes=False),
)
def _gather_kernel(x_hbm, table_hbm, out_hbm, xs, ibuf, bufs, gsem, wsem):
    wid = lax.axis_index("s") * NC + lax.axis_index("c")
    base = wid * ROWS_PER_W
    pltpu.sync_copy(x_hbm.at[pl.ds(wid * B_PER_W, B_PER_W)], xs)

    lane = lax.iota(jnp.int32, 16)

    def compute_idx(j, b):
        for g in range(CHUNK // 16):
            p = j * CHUNK + g * 16 + lane
            i = (p * DIV_M) >> 24
            rem = p - i * QP_PER_I
            jp = rem >> 5
            r = (rem & 31) >> 2
            t = rem & 3
            row = i * 8 + r
            col = 2 * jp + (t >> 1)
            xv = plsc.load_gather(xs, [row, col])
            ibuf[b, pl.ds(g * 16, 16)] = (xv << 2) + jp * (4 * VOCAB) + t

    def gather(b):
        return pltpu.make_async_copy(
            table_hbm.at[ibuf.at[b]], bufs.at[b], gsem.at[b])

    def write(j, b):
        return pltpu.make_async_copy(
            bufs.at[b], out_hbm.at[pl.ds(base + j * CHUNK, CHUNK)], wsem.at[b])

    for b in range(NBUF):
        compute_idx(b, b)
        gather(b).start()

    @pl.loop(0, NCHUNK - NBUF, step=NBUF)
    def _group(j0):
        for b in range(NBUF):
            gather(b).wait()
            write(j0 + b, b).start()
        for b in range(NBUF):
            write(j0 + b, b).wait()
            compute_idx(j0 + b + NBUF, b)
            gather(b).start()

    j0 = NCHUNK - NBUF
    for b in range(NBUF):
        gather(b).wait()
        write(j0 + b, b).start()
    for b in range(NBUF):
        write(j0 + b, b).wait()


def kernel(x_n_cat, tables):
    t13 = jnp.swapaxes(tables, 1, 2).reshape(NPAIR, 2 * HIDDEN, VOCAB)
    tpk = jnp.swapaxes(t13, 1, 2)
    t32 = tpk.reshape(VOCAB * NUM_FIELDS * 2, Q)
    out = _gather_kernel(x_n_cat, t32)
    out4 = out.reshape(BATCH // 8, NPAIR, 8, 2 * HIDDEN)
    return out4.transpose(0, 2, 1, 3).reshape(BATCH, NUM_FIELDS * HIDDEN)
